# baseline (device time: 18534 ns/iter reference)
import jax
import jax.numpy as jnp
from jax import lax
from jax.experimental import pallas as pl
from jax.experimental.pallas import tpu as pltpu

N_Z = 4
M = 512
N_COLS = 2048
CHUNK = N_COLS // N_Z
QW = CHUNK // 4
N_SEG = 4
SM = M // N_SEG


def kernel(x):
    def body(
        x_ref,
        out_ref,
        xq,
        out_vmem,
        send_z,
        recv_z,
        send_xy,
        recv_xy,
        in_sems,
        out_sems,
        send_z_sems,
        recv_z_sems,
        send_xy_sems,
        recv_xy_sems,
    ):
        my_x = lax.axis_index("x")
        my_y = lax.axis_index("y")
        my_z = lax.axis_index("z")
        q = 2 * my_x + my_y

        z_peers = [(my_x, my_y, lax.rem(my_z + dz, N_Z)) for dz in (1, 2, 3)]
        xy_peers = [
            (1 - my_x, my_y, my_z),
            (my_x, 1 - my_y, my_z),
            (1 - my_x, 1 - my_y, my_z),
        ]

        barrier_sem = pltpu.get_barrier_semaphore()
        for pid in z_peers + xy_peers:
            pl.semaphore_signal(
                barrier_sem,
                inc=1,
                device_id=pid,
                device_id_type=pl.DeviceIdType.MESH,
            )

        in_copies = []
        for c in range(N_Z):
            cp = pltpu.make_async_copy(
                x_ref.at[0, :, pl.ds(c * CHUNK + q * QW, QW)],
                xq.at[c],
                in_sems.at[c],
            )
            cp.start()
            in_copies.append(cp)
        for cp in in_copies:
            cp.wait()

        for j, dz in enumerate((1, 2, 3)):
            zt = lax.rem(my_z + dz, N_Z)
            send_z[j, :, :] = xq[zt].astype(jnp.bfloat16)

        pl.semaphore_wait(barrier_sem, 6)

        rows = [slice(seg * SM, (seg + 1) * SM) for seg in range(N_SEG)]

        z_rdmas = [[None] * 3 for _ in range(N_SEG)]
        for seg in range(N_SEG):
            for j, dz in enumerate((1, 2, 3)):
                zt = lax.rem(my_z + dz, N_Z)
                r = pltpu.make_async_remote_copy(
                    src_ref=send_z.at[j, rows[seg]],
                    dst_ref=recv_z.at[2 - j, rows[seg]],
                    send_sem=send_z_sems.at[seg, j],
                    recv_sem=recv_z_sems.at[seg, 2 - j],
                    device_id=(my_x, my_y, zt),
                    device_id_type=pl.DeviceIdType.MESH,
                )
                r.start()
                z_rdmas[seg][j] = r

        xy_rdmas = [[None] * 3 for _ in range(N_SEG)]
        for seg in range(N_SEG):
            for r in z_rdmas[seg]:
                r.wait_recv()
            acc = (
                recv_z[0, rows[seg], :].astype(jnp.float32)
                + recv_z[1, rows[seg], :].astype(jnp.float32)
                + recv_z[2, rows[seg], :].astype(jnp.float32)
                + xq[my_z, rows[seg], :]
            )
            send_xy[rows[seg], :] = acc.astype(jnp.bfloat16)
            for s, pid in enumerate(xy_peers):
                r = pltpu.make_async_remote_copy(
                    src_ref=send_xy.at[rows[seg]],
                    dst_ref=recv_xy.at[s, rows[seg]],
                    send_sem=send_xy_sems.at[seg, s],
                    recv_sem=recv_xy_sems.at[seg, s],
                    device_id=pid,
                    device_id_type=pl.DeviceIdType.MESH,
                )
                r.start()
                xy_rdmas[seg][s] = r
            out_vmem[rows[seg], pl.ds(q * QW, QW)] = acc

        partner_q = [
            2 * (1 - my_x) + my_y,
            2 * my_x + (1 - my_y),
            2 * (1 - my_x) + (1 - my_y),
        ]
        out_copies = []
        for seg in range(N_SEG):
            for s in range(3):
                xy_rdmas[seg][s].wait_recv()
                out_vmem[rows[seg], pl.ds(partner_q[s] * QW, QW)] = recv_xy[
                    s, rows[seg], :
                ].astype(jnp.float32)
            cp = pltpu.make_async_copy(
                out_vmem.at[rows[seg]],
                out_ref.at[rows[seg]],
                out_sems.at[seg],
            )
            cp.start()
            out_copies.append(cp)

        for cp in out_copies:
            cp.wait()
        for seg in range(N_SEG):
            for r in z_rdmas[seg] + xy_rdmas[seg]:
                r.wait_send()

    return pl.pallas_call(
        body,
        out_shape=jax.ShapeDtypeStruct((M, CHUNK), jnp.float32),
        in_specs=[pl.BlockSpec(memory_space=pl.ANY)],
        out_specs=pl.BlockSpec(memory_space=pl.ANY),
        scratch_shapes=[
            pltpu.VMEM((N_Z, M, QW), jnp.float32),
            pltpu.VMEM((M, CHUNK), jnp.float32),
            pltpu.VMEM((3, M, QW), jnp.bfloat16),
            pltpu.VMEM((3, M, QW), jnp.bfloat16),
            pltpu.VMEM((M, QW), jnp.bfloat16),
            pltpu.VMEM((3, M, QW), jnp.bfloat16),
            pltpu.SemaphoreType.DMA((N_Z,)),
            pltpu.SemaphoreType.DMA((N_SEG,)),
            pltpu.SemaphoreType.DMA((N_SEG, 3)),
            pltpu.SemaphoreType.DMA((N_SEG, 3)),
            pltpu.SemaphoreType.DMA((N_SEG, 3)),
            pltpu.SemaphoreType.DMA((N_SEG, 3)),
        ],
        compiler_params=pltpu.CompilerParams(collective_id=0),
    )(x)


# device time: 18163 ns/iter; 1.0204x vs baseline; 1.0204x over previous
import jax
import jax.numpy as jnp
from jax import lax
from jax.experimental import pallas as pl
from jax.experimental.pallas import tpu as pltpu

N_Z = 4
M = 512
N_COLS = 2048
CHUNK = N_COLS // N_Z
QW = CHUNK // 4
N_SEG = 4
SM = M // N_SEG


def kernel(x):
    q_out = 2 * lax.axis_index("x") + lax.axis_index("y")
    xq = jnp.stack(
        [
            lax.dynamic_slice(x, (0, 0, c * CHUNK + q_out * QW), (1, M, QW))[0]
            for c in range(N_Z)
        ]
    )

    def body(
        xq_ref,
        out_ref,
        send_z,
        recv_z,
        send_xy,
        recv_xy,
        send_z_sems,
        recv_z_sems,
        send_xy_sems,
        recv_xy_sems,
    ):
        my_x = lax.axis_index("x")
        my_y = lax.axis_index("y")
        my_z = lax.axis_index("z")
        q = 2 * my_x + my_y

        z_peers = [(my_x, my_y, lax.rem(my_z + dz, N_Z)) for dz in (1, 2, 3)]
        xy_peers = [
            (1 - my_x, my_y, my_z),
            (my_x, 1 - my_y, my_z),
            (1 - my_x, 1 - my_y, my_z),
        ]

        barrier_sem = pltpu.get_barrier_semaphore()
        for pid in z_peers + xy_peers:
            pl.semaphore_signal(
                barrier_sem,
                inc=1,
                device_id=pid,
                device_id_type=pl.DeviceIdType.MESH,
            )

        for j, dz in enumerate((1, 2, 3)):
            zt = lax.rem(my_z + dz, N_Z)
            send_z[j, :, :] = xq_ref[zt].astype(jnp.bfloat16)

        pl.semaphore_wait(barrier_sem, 6)

        rows = [slice(seg * SM, (seg + 1) * SM) for seg in range(N_SEG)]

        z_rdmas = [[None] * 3 for _ in range(N_SEG)]
        for seg in range(N_SEG):
            for j, dz in enumerate((1, 2, 3)):
                zt = lax.rem(my_z + dz, N_Z)
                r = pltpu.make_async_remote_copy(
                    src_ref=send_z.at[j, rows[seg]],
                    dst_ref=recv_z.at[2 - j, rows[seg]],
                    send_sem=send_z_sems.at[seg, j],
                    recv_sem=recv_z_sems.at[seg, 2 - j],
                    device_id=(my_x, my_y, zt),
                    device_id_type=pl.DeviceIdType.MESH,
                )
                r.start()
                z_rdmas[seg][j] = r

        xy_rdmas = [[None] * 3 for _ in range(N_SEG)]
        for seg in range(N_SEG):
            for r in z_rdmas[seg]:
                r.wait_recv()
            acc = (
                recv_z[0, rows[seg], :].astype(jnp.float32)
                + recv_z[1, rows[seg], :].astype(jnp.float32)
                + recv_z[2, rows[seg], :].astype(jnp.float32)
                + xq_ref[my_z, rows[seg], :]
            )
            send_xy[rows[seg], :] = acc.astype(jnp.bfloat16)
            for s, pid in enumerate(xy_peers):
                r = pltpu.make_async_remote_copy(
                    src_ref=send_xy.at[rows[seg]],
                    dst_ref=recv_xy.at[s, rows[seg]],
                    send_sem=send_xy_sems.at[seg, s],
                    recv_sem=recv_xy_sems.at[seg, s],
                    device_id=pid,
                    device_id_type=pl.DeviceIdType.MESH,
                )
                r.start()
                xy_rdmas[seg][s] = r
            out_ref[rows[seg], pl.ds(q * QW, QW)] = acc

        partner_q = [
            2 * (1 - my_x) + my_y,
            2 * my_x + (1 - my_y),
            2 * (1 - my_x) + (1 - my_y),
        ]
        for seg in range(N_SEG):
            for s in range(3):
                xy_rdmas[seg][s].wait_recv()
                out_ref[rows[seg], pl.ds(partner_q[s] * QW, QW)] = recv_xy[
                    s, rows[seg], :
                ].astype(jnp.float32)

        for seg in range(N_SEG):
            for r in z_rdmas[seg] + xy_rdmas[seg]:
                r.wait_send()

    return pl.pallas_call(
        body,
        out_shape=jax.ShapeDtypeStruct((M, CHUNK), jnp.float32),
        in_specs=[pl.BlockSpec(memory_space=pltpu.VMEM)],
        out_specs=pl.BlockSpec(memory_space=pltpu.VMEM),
        scratch_shapes=[
            pltpu.VMEM((3, M, QW), jnp.bfloat16),
            pltpu.VMEM((3, M, QW), jnp.bfloat16),
            pltpu.VMEM((M, QW), jnp.bfloat16),
            pltpu.VMEM((3, M, QW), jnp.bfloat16),
            pltpu.SemaphoreType.DMA((N_SEG, 3)),
            pltpu.SemaphoreType.DMA((N_SEG, 3)),
            pltpu.SemaphoreType.DMA((N_SEG, 3)),
            pltpu.SemaphoreType.DMA((N_SEG, 3)),
        ],
        compiler_params=pltpu.CompilerParams(collective_id=0),
    )(xq)
